# baseline (speedup 1.0000x reference)
import jax
import jax.numpy as jnp
from jax import lax
from jax.experimental import pallas as pl
from jax.experimental.pallas import tpu as pltpu
from jax.experimental.pallas import tpu_sc as plsc

B = 16384
D = 64
NC = 2
NS = 16
L = 16
NW = NC * NS
BPW = B // NW
NGROUP = BPW // L
TILE_W = 128
NSLOT = 4


def _tt_body(dids_hbm, rids_hbm, dtab_hbm, rtab_hbm, out_hbm,
             dids_s, rids_s, dblk_v, rblk_v, dstage_v, rstage_v,
             out_v, sem_d0, sem_d1, sem_r):
    wid = lax.axis_index("s") * NC + lax.axis_index("c")
    base = wid * BPW

    pltpu.sync_copy(dids_hbm.at[wid], dids_s)
    pltpu.sync_copy(rids_hbm.at[wid], rids_s)

    lane = lax.iota(jnp.int32, L)
    dsems = [sem_d0, sem_d1]

    def fire_donor(dgrp, q, buf):
        for s in range(NSLOT):
            did = dgrp[q * NSLOT + s]
            dstart = pl.multiple_of((did // TILE_W) * TILE_W, TILE_W)
            pltpu.async_copy(dtab_hbm.at[:, pl.ds(dstart, TILE_W)],
                             dblk_v.at[buf, s], dsems[buf])

    def group_body(g, carry):
        dgrp = dids_s[pl.ds(g * L, L)]
        rgrp = rids_s[pl.ds(g * L, L)]
        fire_donor(dgrp, 0, 0)
        for q in range(L // NSLOT):
            buf = q % 2
            if q + 1 < L // NSLOT:
                fire_donor(dgrp, q + 1, 1 - buf)
            rids_q = []
            for s in range(NSLOT):
                rid = rgrp[q * NSLOT + s]
                rstart = pl.multiple_of((rid // TILE_W) * TILE_W, TILE_W)
                rids_q.append(rid)
                pltpu.async_copy(
                    rtab_hbm.at[:, pl.ds(rstart, TILE_W)], rblk_v.at[s], sem_r)
            for s in range(NSLOT):
                pltpu.make_async_copy(dtab_hbm.at[:, pl.ds(0, TILE_W)],
                                      dblk_v.at[buf, s], dsems[buf]).wait()
                pltpu.make_async_copy(rtab_hbm.at[:, pl.ds(0, TILE_W)],
                                      rblk_v.at[s], sem_r).wait()
            for s in range(NSLOT):
                b_local = q * NSLOT + s
                did = dgrp[b_local]
                rid = rids_q[s]
                dj = jnp.full((L,), lax.rem(did, TILE_W), jnp.int32)
                rj = jnp.full((L,), lax.rem(rid, TILE_W), jnp.int32)
                bufv = jnp.full((L,), buf, jnp.int32)
                slot = jnp.full((L,), s, jnp.int32)
                for k in range(D // L):
                    drow = plsc.load_gather(dblk_v,
                                            [bufv, slot, lane + k * L, dj])
                    rrow = plsc.load_gather(rblk_v, [slot, lane + k * L, rj])
                    dstage_v.at[b_local][pl.ds(k * L, L)] = drow
                    rstage_v.at[b_local][pl.ds(k * L, L)] = rrow

        acc = jnp.zeros((L,), jnp.float32)
        for d in range(D):
            dcol = jnp.full((L,), d, jnp.int32)
            a = plsc.load_gather(dstage_v, [lane, dcol])
            b = plsc.load_gather(rstage_v, [lane, dcol])
            acc = acc + a * b
        out_v[pl.ds(g * L, L)] = acc
        return carry

    lax.fori_loop(0, NGROUP, group_body, 0)

    pltpu.sync_copy(out_v, out_hbm.at[pl.ds(base, BPW)])


def kernel(donor_ids, receiver_ids, donor_table, receiver_table):
    dids = donor_ids.astype(jnp.int32).reshape(NW, BPW)
    rids = receiver_ids.astype(jnp.int32).reshape(NW, BPW)

    mesh = plsc.VectorSubcoreMesh(core_axis_name="c", subcore_axis_name="s",
                                  num_cores=NC, num_subcores=NS)
    run = pl.kernel(
        _tt_body,
        out_type=jax.ShapeDtypeStruct((B,), jnp.float32),
        mesh=mesh,
        compiler_params=pltpu.CompilerParams(needs_layout_passes=False),
        scratch_types=[
            pltpu.VMEM((BPW,), jnp.int32),
            pltpu.VMEM((BPW,), jnp.int32),
            pltpu.VMEM((2, NSLOT, D, TILE_W), jnp.float32),
            pltpu.VMEM((NSLOT, D, TILE_W), jnp.float32),
            pltpu.VMEM((L, D), jnp.float32),
            pltpu.VMEM((L, D), jnp.float32),
            pltpu.VMEM((BPW,), jnp.float32),
            pltpu.SemaphoreType.DMA,
            pltpu.SemaphoreType.DMA,
            pltpu.SemaphoreType.DMA,
        ],
    )
    return run(dids, rids, donor_table.T, receiver_table.T)


# speedup vs baseline: 1.0077x; 1.0077x over previous
import jax
import jax.numpy as jnp
from jax import lax
from jax.experimental import pallas as pl
from jax.experimental.pallas import tpu as pltpu
from jax.experimental.pallas import tpu_sc as plsc

B = 16384
D = 64
NC = 2
NS = 16
L = 16
NW = NC * NS
BPW = B // NW
NGROUP = BPW // L
TILE_W = 128
NSLOT = 4


def _tt_body(dids_hbm, rids_hbm, dtab_hbm, rtab_hbm, out_hbm,
             dids_s, rids_s, dblk_v, rblk_v, dstage_v, rstage_v,
             out_v, sem, sem_r):
    wid = lax.axis_index("s") * NC + lax.axis_index("c")
    base = wid * BPW

    pltpu.sync_copy(dids_hbm.at[wid], dids_s)
    pltpu.sync_copy(rids_hbm.at[wid], rids_s)

    lane = lax.iota(jnp.int32, L)

    def group_body(g, carry):
        dgrp = dids_s[pl.ds(g * L, L)]
        rgrp = rids_s[pl.ds(g * L, L)]
        for q in range(L // NSLOT):
            dcopies, rcopies = [], []
            ids_q = []
            for s in range(NSLOT):
                b_local = q * NSLOT + s
                did = dgrp[b_local]
                rid = rgrp[b_local]
                dstart = pl.multiple_of((did // TILE_W) * TILE_W, TILE_W)
                rstart = pl.multiple_of((rid // TILE_W) * TILE_W, TILE_W)
                ids_q.append((did, rid))
                dcopies.append(pltpu.async_copy(
                    dtab_hbm.at[:, pl.ds(dstart, TILE_W)], dblk_v.at[s], sem))
                rcopies.append(pltpu.async_copy(
                    rtab_hbm.at[:, pl.ds(rstart, TILE_W)], rblk_v.at[s], sem_r))
            for cp in dcopies:
                cp.wait()
            for s in range(NSLOT):
                b_local = q * NSLOT + s
                did, _ = ids_q[s]
                dj = jnp.full((L,), lax.rem(did, TILE_W), jnp.int32)
                slot = jnp.full((L,), s, jnp.int32)
                for k in range(D // L):
                    drow = plsc.load_gather(dblk_v, [slot, lane + k * L, dj])
                    dstage_v.at[b_local][pl.ds(k * L, L)] = drow
            for cp in rcopies:
                cp.wait()
            for s in range(NSLOT):
                b_local = q * NSLOT + s
                _, rid = ids_q[s]
                rj = jnp.full((L,), lax.rem(rid, TILE_W), jnp.int32)
                slot = jnp.full((L,), s, jnp.int32)
                for k in range(D // L):
                    rrow = plsc.load_gather(rblk_v, [slot, lane + k * L, rj])
                    rstage_v.at[b_local][pl.ds(k * L, L)] = rrow

        acc = jnp.zeros((L,), jnp.float32)
        for d in range(D):
            dcol = jnp.full((L,), d, jnp.int32)
            a = plsc.load_gather(dstage_v, [lane, dcol])
            b = plsc.load_gather(rstage_v, [lane, dcol])
            acc = acc + a * b
        out_v[pl.ds(g * L, L)] = acc
        return carry

    lax.fori_loop(0, NGROUP, group_body, 0)

    pltpu.sync_copy(out_v, out_hbm.at[pl.ds(base, BPW)])


def kernel(donor_ids, receiver_ids, donor_table, receiver_table):
    dids = donor_ids.astype(jnp.int32).reshape(NW, BPW)
    rids = receiver_ids.astype(jnp.int32).reshape(NW, BPW)

    mesh = plsc.VectorSubcoreMesh(core_axis_name="c", subcore_axis_name="s",
                                  num_cores=NC, num_subcores=NS)
    run = pl.kernel(
        _tt_body,
        out_type=jax.ShapeDtypeStruct((B,), jnp.float32),
        mesh=mesh,
        compiler_params=pltpu.CompilerParams(needs_layout_passes=False),
        scratch_types=[
            pltpu.VMEM((BPW,), jnp.int32),
            pltpu.VMEM((BPW,), jnp.int32),
            pltpu.VMEM((NSLOT, D, TILE_W), jnp.float32),
            pltpu.VMEM((NSLOT, D, TILE_W), jnp.float32),
            pltpu.VMEM((L, D), jnp.float32),
            pltpu.VMEM((L, D), jnp.float32),
            pltpu.VMEM((BPW,), jnp.float32),
            pltpu.SemaphoreType.DMA,
            pltpu.SemaphoreType.DMA,
        ],
    )
    return run(dids, rids, donor_table.T, receiver_table.T)


# speedup vs baseline: 1.0600x; 1.0519x over previous
import jax
import jax.numpy as jnp
from jax import lax
from jax.experimental import pallas as pl
from jax.experimental.pallas import tpu as pltpu
from jax.experimental.pallas import tpu_sc as plsc

B = 16384
D = 64
NC = 2
NS = 16
L = 16
NW = NC * NS
BPW = B // NW
NGROUP = BPW // L
TILE_W = 128
NSLOT = 4


def _tt_body(dids_hbm, rids_hbm, dtab_hbm, rtab_hbm, out_hbm,
             dids_s, rids_s, dblk_v, rblk_v, dstage_v, rstage_v,
             out_v, sem):
    wid = lax.axis_index("s") * NC + lax.axis_index("c")
    base = wid * BPW

    pltpu.sync_copy(dids_hbm.at[wid], dids_s)
    pltpu.sync_copy(rids_hbm.at[wid], rids_s)

    lane = lax.iota(jnp.int32, L)

    def group_body(g, carry):
        dgrp = dids_s[pl.ds(g * L, L)]
        rgrp = rids_s[pl.ds(g * L, L)]
        for q in range(L // NSLOT):
            copies = []
            ids_q = []
            for s in range(NSLOT):
                b_local = q * NSLOT + s
                did = dgrp[b_local]
                rid = rgrp[b_local]
                dstart = pl.multiple_of((did // TILE_W) * TILE_W, TILE_W)
                rstart = pl.multiple_of((rid // TILE_W) * TILE_W, TILE_W)
                ids_q.append((did, rid))
                copies.append(pltpu.async_copy(
                    dtab_hbm.at[:, pl.ds(dstart, TILE_W)], dblk_v.at[s], sem))
                copies.append(pltpu.async_copy(
                    rtab_hbm.at[:, pl.ds(rstart, TILE_W)], rblk_v.at[s], sem))
            for cp in copies:
                cp.wait()
            for s in range(NSLOT):
                b_local = q * NSLOT + s
                did, rid = ids_q[s]
                dj = jnp.full((L,), lax.rem(did, TILE_W), jnp.int32)
                rj = jnp.full((L,), lax.rem(rid, TILE_W), jnp.int32)
                slot = jnp.full((L,), s, jnp.int32)
                for k in range(D // L):
                    drow = plsc.load_gather(dblk_v, [slot, lane + k * L, dj])
                    rrow = plsc.load_gather(rblk_v, [slot, lane + k * L, rj])
                    dstage_v.at[b_local][pl.ds(k * L, L)] = drow
                    rstage_v.at[b_local][pl.ds(k * L, L)] = rrow

        acc = jnp.zeros((L,), jnp.float32)
        for d in range(D):
            dcol = jnp.full((L,), d, jnp.int32)
            a = plsc.load_gather(dstage_v, [lane, dcol])
            b = plsc.load_gather(rstage_v, [lane, dcol])
            acc = acc + a * b
        out_v[pl.ds(g * L, L)] = acc
        return carry

    lax.fori_loop(0, NGROUP, group_body, 0)

    pltpu.sync_copy(out_v, out_hbm.at[pl.ds(base, BPW)])


def kernel(donor_ids, receiver_ids, donor_table, receiver_table):
    dids = donor_ids.astype(jnp.int32).reshape(NW, BPW)
    rids = receiver_ids.astype(jnp.int32).reshape(NW, BPW)

    mesh = plsc.VectorSubcoreMesh(core_axis_name="c", subcore_axis_name="s",
                                  num_cores=NC, num_subcores=NS)
    run = pl.kernel(
        _tt_body,
        out_type=jax.ShapeDtypeStruct((B,), jnp.float32),
        mesh=mesh,
        compiler_params=pltpu.CompilerParams(needs_layout_passes=False),
        scratch_types=[
            pltpu.VMEM((BPW,), jnp.int32),
            pltpu.VMEM((BPW,), jnp.int32),
            pltpu.VMEM((NSLOT, D, TILE_W), jnp.float32),
            pltpu.VMEM((NSLOT, D, TILE_W), jnp.float32),
            pltpu.VMEM((L, D), jnp.float32),
            pltpu.VMEM((L, D), jnp.float32),
            pltpu.VMEM((BPW,), jnp.float32),
            pltpu.SemaphoreType.DMA,
        ],
    )
    return run(dids, rids, donor_table.T, receiver_table.T)
